# Initial kernel scaffold; baseline (speedup 1.0000x reference)
import jax
import jax.numpy as jnp
from jax.experimental import pallas as pl


def kernel(x, edge_index, pos, batch_index, W_gate, b_gate, W_msg, b_msg):
    raise NotImplementedError("write your pallas kernel here")


# speedup vs baseline: 8.0495x; 8.0495x over previous
import functools

import jax
import jax.numpy as jnp
from jax.experimental import pallas as pl
from jax.experimental.pallas import tpu as pltpu

_NUM_SEGMENTS = 256


def _tc_body(G, Nb, B, N, x_ref, pos_ref, bid_ref, wg_ref, bg_ref, wm_ref,
             bm_ref, out_ref, acc_ref, asum_ref):
    i = pl.program_id(0)

    @pl.when(i == 0)
    def _():
        acc_ref[...] = jnp.zeros_like(acc_ref)
        asum_ref[...] = jnp.zeros_like(asum_ref)

    xb = x_ref[...]
    a = jax.lax.dot_general(xb, wg_ref[...], (((1,), (0,)), ((), ())),
                            preferred_element_type=jnp.float32)
    a = a + bg_ref[...]
    alpha = pos_ref[0] * jnp.exp(a)
    if N % Nb != 0:
        valid = (jax.lax.broadcasted_iota(jnp.int32, (Nb, 1), 0)
                 + i * Nb) < N
        alpha = jnp.where(valid, alpha, 0.0)
    beta = jax.lax.dot_general(xb, wm_ref[...], (((1,), (0,)), ((), ())),
                               preferred_element_type=jnp.float32)
    beta = beta + bm_ref[...]
    w = alpha * beta
    if N % Nb != 0:
        w = jnp.where(valid, w, 0.0)
    bid = bid_ref[0]
    onehot_t = (jax.lax.broadcasted_iota(jnp.int32, (B, Nb), 0)
                == bid).astype(jnp.float32)
    acc_ref[...] += jax.lax.dot_general(
        onehot_t, w, (((1,), (0,)), ((), ())),
        preferred_element_type=jnp.float32)
    asum_ref[...] += jax.lax.dot_general(
        onehot_t, alpha, (((1,), (0,)), ((), ())),
        preferred_element_type=jnp.float32)

    @pl.when(i == G - 1)
    def _():
        denom = asum_ref[...]
        denom = jnp.where(denom == 0.0, 1.0, denom)
        out_ref[...] = acc_ref[...] / denom


def kernel(x, edge_index, pos, batch_index, W_gate, b_gate, W_msg, b_msg):
    del edge_index
    N, D = x.shape
    B = _NUM_SEGMENTS
    Nb = 2000 if N % 2000 == 0 else 1024
    G = -(-N // Nb)
    pad = G * Nb - N
    if pad:
        pos = jnp.pad(pos, (0, pad))
        batch_index = jnp.pad(batch_index, (0, pad))
    pos3 = pos.reshape(G, Nb, 1)
    bid3 = batch_index.reshape(G, 1, Nb)
    bg2 = b_gate.reshape(1, 1)
    bm2 = b_msg.reshape(1, D)
    out = pl.pallas_call(
        functools.partial(_tc_body, G, Nb, B, N),
        grid=(G,),
        in_specs=[
            pl.BlockSpec((Nb, D), lambda i: (i, 0)),
            pl.BlockSpec((1, Nb, 1), lambda i: (i, 0, 0)),
            pl.BlockSpec((1, 1, Nb), lambda i: (i, 0, 0)),
            pl.BlockSpec((D, 1), lambda i: (0, 0)),
            pl.BlockSpec((1, 1), lambda i: (0, 0)),
            pl.BlockSpec((D, D), lambda i: (0, 0)),
            pl.BlockSpec((1, D), lambda i: (0, 0)),
        ],
        out_specs=pl.BlockSpec((B, D), lambda i: (0, 0)),
        out_shape=jax.ShapeDtypeStruct((B, D), jnp.float32),
        scratch_shapes=[pltpu.VMEM((B, D), jnp.float32),
                        pltpu.VMEM((B, 1), jnp.float32)],
    )(x, pos3, bid3, W_gate, bg2, W_msg, bm2)
    return out


# speedup vs baseline: 17.5989x; 2.1863x over previous
import functools

import jax
import jax.numpy as jnp
from jax.experimental import pallas as pl
from jax.experimental.pallas import tpu as pltpu

_NUM_SEGMENTS = 256
_WIN = 32


def _tc_body(G, Nb, B, N, S, bases_ref, lasts_ref, x_ref, pos_ref, bid_ref,
             wg_ref, bg_ref, wm_ref, bm_ref, out_ref, acc_ref, asum_ref):
    i = pl.program_id(0)

    @pl.when(i == 0)
    def _():
        acc_ref[...] = jnp.zeros_like(acc_ref)
        asum_ref[...] = jnp.zeros_like(asum_ref)

    xb = x_ref[...]
    if N % Nb != 0:
        valid_col = (jax.lax.broadcasted_iota(jnp.int32, (Nb, 1), 0)
                     + i * Nb) < N
        xb = jnp.where(valid_col, xb, 0.0)
    a_t = jax.lax.dot_general(wg_ref[...], xb, (((0,), (1,)), ((), ())),
                              preferred_element_type=jnp.float32)
    alpha_t = pos_ref[0] * jnp.exp(a_t + bg_ref[...])
    if N % Nb != 0:
        valid = (jax.lax.broadcasted_iota(jnp.int32, (1, Nb), 1)
                 + i * Nb) < N
        alpha_t = jnp.where(valid, alpha_t, 0.0)
    bid = bid_ref[0]
    base = bases_ref[i]
    small = (lasts_ref[i] - base) < S

    @pl.when(small)
    def _():
        sel = (jax.lax.broadcasted_iota(jnp.int32, (S, Nb), 0) + base) == bid
        ohw = jnp.where(sel, alpha_t, 0.0)
        acc_ref[pl.ds(base, S), :] += jax.lax.dot_general(
            ohw, xb, (((1,), (0,)), ((), ())),
            preferred_element_type=jnp.float32)
        asum_ref[pl.ds(base, S), :] += jnp.sum(ohw, axis=1, keepdims=True)

    @pl.when(jnp.logical_not(small))
    def _():
        sel = jax.lax.broadcasted_iota(jnp.int32, (B, Nb), 0) == bid
        ohw = jnp.where(sel, alpha_t, 0.0)
        acc_ref[...] += jax.lax.dot_general(
            ohw, xb, (((1,), (0,)), ((), ())),
            preferred_element_type=jnp.float32)
        asum_ref[...] += jnp.sum(ohw, axis=1, keepdims=True)

    @pl.when(i == G - 1)
    def _():
        denom = asum_ref[...]
        denom = jnp.where(denom == 0.0, 1.0, denom)
        pooled = jax.lax.dot_general(
            acc_ref[...], wm_ref[...], (((1,), (0,)), ((), ())),
            preferred_element_type=jnp.float32)
        out_ref[...] = pooled / denom + bm_ref[...]


def kernel(x, edge_index, pos, batch_index, W_gate, b_gate, W_msg, b_msg):
    del edge_index
    N, D = x.shape
    B = _NUM_SEGMENTS
    S = _WIN
    Nb = 2000 if N % 2000 == 0 else 1024
    G = -(-N // Nb)
    pad = G * Nb - N
    if pad:
        pos = jnp.pad(pos, (0, pad))
        batch_index = jnp.pad(batch_index, (0, pad), mode="edge")
    bid_r = batch_index.reshape(G, Nb)
    firsts = bid_r[:, 0]
    lasts = bid_r[:, -1]
    bases = jnp.minimum((firsts // 8) * 8, B - S)
    pos3 = pos.reshape(G, 1, Nb)
    bid3 = batch_index.reshape(G, 1, Nb)
    bg2 = b_gate.reshape(1, 1)
    bm2 = b_msg.reshape(1, D)
    out = pl.pallas_call(
        functools.partial(_tc_body, G, Nb, B, N, S),
        grid=(G,),
        in_specs=[
            pl.BlockSpec(memory_space=pltpu.SMEM),
            pl.BlockSpec(memory_space=pltpu.SMEM),
            pl.BlockSpec((Nb, D), lambda i: (i, 0)),
            pl.BlockSpec((1, 1, Nb), lambda i: (i, 0, 0)),
            pl.BlockSpec((1, 1, Nb), lambda i: (i, 0, 0)),
            pl.BlockSpec((D, 1), lambda i: (0, 0)),
            pl.BlockSpec((1, 1), lambda i: (0, 0)),
            pl.BlockSpec((D, D), lambda i: (0, 0)),
            pl.BlockSpec((1, D), lambda i: (0, 0)),
        ],
        out_specs=pl.BlockSpec((B, D), lambda i: (0, 0)),
        out_shape=jax.ShapeDtypeStruct((B, D), jnp.float32),
        scratch_shapes=[pltpu.VMEM((B, D), jnp.float32),
                        pltpu.VMEM((B, 1), jnp.float32)],
    )(bases, lasts, x, pos3, bid3, W_gate, bg2, W_msg, bm2)
    return out


# speedup vs baseline: 18.3944x; 1.0452x over previous
import functools

import jax
import jax.numpy as jnp
from jax.experimental import pallas as pl
from jax.experimental.pallas import tpu as pltpu

_NUM_SEGMENTS = 256
_WIN = 32


def _body(G, Nb, B, N, S, bases_ref, x_ref, pos_ref, bid_ref,
          wg_ref, bg_ref, wm_ref, bm_ref, out_ref, acc_ref, asum_ref):
    i = pl.program_id(0)

    @pl.when(i == 0)
    def _():
        acc_ref[...] = jnp.zeros_like(acc_ref)
        asum_ref[...] = jnp.zeros_like(asum_ref)

    xb = x_ref[...]
    if N % Nb != 0:
        valid_col = (jax.lax.broadcasted_iota(jnp.int32, (Nb, 1), 0)
                     + i * Nb) < N
        xb = jnp.where(valid_col, xb, 0.0)
    a_t = jax.lax.dot_general(wg_ref[...], xb, (((0,), (1,)), ((), ())),
                              preferred_element_type=jnp.float32)
    alpha_t = pos_ref[0] * jnp.exp(a_t + bg_ref[...])
    if N % Nb != 0:
        valid = (jax.lax.broadcasted_iota(jnp.int32, (1, Nb), 1)
                 + i * Nb) < N
        alpha_t = jnp.where(valid, alpha_t, 0.0)
    bid = bid_ref[0]
    base = bases_ref[i] if S < B else 0
    sel = (jax.lax.broadcasted_iota(jnp.int32, (S, Nb), 0) + base) == bid
    ohw = jnp.where(sel, alpha_t, 0.0)
    upd = jax.lax.dot_general(ohw, xb, (((1,), (0,)), ((), ())),
                              preferred_element_type=jnp.float32)
    acc_ref[pl.ds(base, S), :] += upd
    asum_ref[pl.ds(base, S), :] += jnp.sum(ohw, axis=1, keepdims=True)

    @pl.when(i == G - 1)
    def _():
        denom = asum_ref[...]
        denom = jnp.where(denom == 0.0, 1.0, denom)
        pooled = jax.lax.dot_general(
            acc_ref[...], wm_ref[...], (((1,), (0,)), ((), ())),
            preferred_element_type=jnp.float32)
        out_ref[...] = pooled / denom + bm_ref[...]


def _make_call(G, Nb, B, N, S, D):
    return pl.pallas_call(
        functools.partial(_body, G, Nb, B, N, S),
        grid=(G,),
        in_specs=[
            pl.BlockSpec(memory_space=pltpu.SMEM),
            pl.BlockSpec((Nb, D), lambda i: (i, 0)),
            pl.BlockSpec((1, 1, Nb), lambda i: (i, 0, 0)),
            pl.BlockSpec((1, 1, Nb), lambda i: (i, 0, 0)),
            pl.BlockSpec((D, 1), lambda i: (0, 0)),
            pl.BlockSpec((1, 1), lambda i: (0, 0)),
            pl.BlockSpec((D, D), lambda i: (0, 0)),
            pl.BlockSpec((1, D), lambda i: (0, 0)),
        ],
        out_specs=pl.BlockSpec((B, D), lambda i: (0, 0)),
        out_shape=jax.ShapeDtypeStruct((B, D), jnp.float32),
        scratch_shapes=[pltpu.VMEM((B, D), jnp.float32),
                        pltpu.VMEM((B, 1), jnp.float32)],
    )


def kernel(x, edge_index, pos, batch_index, W_gate, b_gate, W_msg, b_msg):
    del edge_index
    N, D = x.shape
    B = _NUM_SEGMENTS
    S = _WIN
    Nb = 2000 if N % 2000 == 0 else 1024
    G = -(-N // Nb)
    pad = G * Nb - N
    if pad:
        pos = jnp.pad(pos, (0, pad))
        batch_index = jnp.pad(batch_index, (0, pad), mode="edge")
    bid_r = batch_index.reshape(G, Nb)
    firsts = bid_r[:, 0]
    lasts = bid_r[:, -1]
    bases = jnp.minimum((firsts // 8) * 8, B - S)
    all_small = jnp.all(lasts - bases < S)
    pos3 = pos.reshape(G, 1, Nb)
    bid3 = batch_index.reshape(G, 1, Nb)
    bg2 = b_gate.reshape(1, 1)
    bm2 = b_msg.reshape(1, D)
    ops = (bases, x, pos3, bid3, W_gate, bg2, W_msg, bm2)
    return jax.lax.cond(
        all_small,
        lambda o: _make_call(G, Nb, B, N, S, D)(*o),
        lambda o: _make_call(G, Nb, B, N, B, D)(*o),
        ops)


# speedup vs baseline: 26.0177x; 1.4144x over previous
import functools

import jax
import jax.numpy as jnp
from jax.experimental import pallas as pl
from jax.experimental.pallas import tpu as pltpu

_NUM_SEGMENTS = 256
_WIN = 32


def _body(G, Nb, B, N, S, bases_ref, x_ref, pos_ref, bid_ref,
          wg_ref, bg_ref, wm_ref, bm_ref, out_ref, acc_ref, asum_ref):
    i = pl.program_id(0)

    @pl.when(i == 0)
    def _():
        acc_ref[...] = jnp.zeros_like(acc_ref)
        asum_ref[...] = jnp.zeros_like(asum_ref)

    xb = x_ref[...]
    if N % Nb != 0:
        valid_col = (jax.lax.broadcasted_iota(jnp.int32, (Nb, 1), 0)
                     + i * Nb) < N
        xb = jnp.where(valid_col, xb, 0.0)
    a_t = jax.lax.dot_general(wg_ref[...], xb, (((0,), (1,)), ((), ())),
                              preferred_element_type=jnp.float32)
    alpha_t = pos_ref[0] * jnp.exp(a_t + bg_ref[...])
    if N % Nb != 0:
        valid = (jax.lax.broadcasted_iota(jnp.int32, (1, Nb), 1)
                 + i * Nb) < N
        alpha_t = jnp.where(valid, alpha_t, 0.0)
    bid = bid_ref[0]
    base = bases_ref[i] if S < B else 0
    sel = (jax.lax.broadcasted_iota(jnp.int32, (S, Nb), 0) + base) == bid
    ohw = jnp.where(sel, alpha_t, 0.0)
    upd = jax.lax.dot_general(ohw, xb, (((1,), (0,)), ((), ())),
                              preferred_element_type=jnp.float32)
    acc_ref[pl.ds(base, S), :] += upd
    asum_ref[pl.ds(base, S), :] += jnp.sum(ohw, axis=1, keepdims=True)

    @pl.when(i == G - 1)
    def _():
        denom = asum_ref[...]
        denom = jnp.where(denom == 0.0, 1.0, denom)
        pooled = jax.lax.dot_general(
            acc_ref[...], wm_ref[...], (((1,), (0,)), ((), ())),
            preferred_element_type=jnp.float32)
        out_ref[...] = pooled / denom + bm_ref[...]


def _make_call(G, Nb, B, N, S, D):
    return pl.pallas_call(
        functools.partial(_body, G, Nb, B, N, S),
        grid=(G,),
        in_specs=[
            pl.BlockSpec(memory_space=pltpu.SMEM),
            pl.BlockSpec((Nb, D), lambda i: (i, 0)),
            pl.BlockSpec((1, 1, Nb), lambda i: (i, 0, 0)),
            pl.BlockSpec((1, 1, Nb), lambda i: (i, 0, 0)),
            pl.BlockSpec((D, 1), lambda i: (0, 0)),
            pl.BlockSpec((1, 1), lambda i: (0, 0)),
            pl.BlockSpec((D, D), lambda i: (0, 0)),
            pl.BlockSpec((1, D), lambda i: (0, 0)),
        ],
        out_specs=pl.BlockSpec((B, D), lambda i: (0, 0)),
        out_shape=jax.ShapeDtypeStruct((B, D), jnp.float32),
        scratch_shapes=[pltpu.VMEM((B, D), jnp.float32),
                        pltpu.VMEM((B, 1), jnp.float32)],
    )


def kernel(x, edge_index, pos, batch_index, W_gate, b_gate, W_msg, b_msg):
    del edge_index
    N, D = x.shape
    B = _NUM_SEGMENTS
    S = _WIN
    Nb = 4000 if N % 4000 == 0 else 1024
    G = -(-N // Nb)
    pad = G * Nb - N
    if pad:
        pos = jnp.pad(pos, (0, pad))
        batch_index = jnp.pad(batch_index, (0, pad), mode="edge")
    bid_r = batch_index.reshape(G, Nb)
    firsts = bid_r[:, 0]
    lasts = bid_r[:, -1]
    bases = jnp.minimum((firsts // 8) * 8, B - S)
    all_small = jnp.all(lasts - bases < S)
    pos3 = pos.reshape(G, 1, Nb)
    bid3 = batch_index.reshape(G, 1, Nb)
    bg2 = b_gate.reshape(1, 1)
    bm2 = b_msg.reshape(1, D)
    ops = (bases, x, pos3, bid3, W_gate, bg2, W_msg, bm2)
    return jax.lax.cond(
        all_small,
        lambda o: _make_call(G, Nb, B, N, S, D)(*o),
        lambda o: _make_call(G, Nb, B, N, B, D)(*o),
        ops)


# speedup vs baseline: 28.4133x; 1.0921x over previous
import functools

import jax
import jax.numpy as jnp
from jax.experimental import pallas as pl
from jax.experimental.pallas import tpu as pltpu

_NUM_SEGMENTS = 256
_WIN = 32


def _body(G, Nb, B, N, S, bases_ref, x_ref, pos_ref, bid_ref,
          wg_ref, bg_ref, wm_ref, bm_ref, out_ref, acc_ref, asum_ref):
    i = pl.program_id(0)

    @pl.when(i == 0)
    def _():
        acc_ref[...] = jnp.zeros_like(acc_ref)
        asum_ref[...] = jnp.zeros_like(asum_ref)

    xb = x_ref[...]
    if N % Nb != 0:
        valid_col = (jax.lax.broadcasted_iota(jnp.int32, (Nb, 1), 0)
                     + i * Nb) < N
        xb = jnp.where(valid_col, xb, 0.0)
    a_t = jax.lax.dot_general(wg_ref[...], xb, (((0,), (1,)), ((), ())),
                              preferred_element_type=jnp.float32)
    alpha_t = pos_ref[0] * jnp.exp(a_t + bg_ref[...])
    if N % Nb != 0:
        valid = (jax.lax.broadcasted_iota(jnp.int32, (1, Nb), 1)
                 + i * Nb) < N
        alpha_t = jnp.where(valid, alpha_t, 0.0)
    bid = bid_ref[0]
    base = bases_ref[i] if S < B else 0
    sel = (jax.lax.broadcasted_iota(jnp.int32, (S, Nb), 0) + base) == bid
    ohw = jnp.where(sel, alpha_t, 0.0)
    upd = jax.lax.dot_general(ohw, xb, (((1,), (0,)), ((), ())),
                              preferred_element_type=jnp.float32)
    acc_ref[pl.ds(base, S), :] += upd
    asum_ref[pl.ds(base, S), :] += jnp.sum(ohw, axis=1, keepdims=True)

    @pl.when(i == G - 1)
    def _():
        denom = asum_ref[...]
        denom = jnp.where(denom == 0.0, 1.0, denom)
        pooled = jax.lax.dot_general(
            acc_ref[...], wm_ref[...], (((1,), (0,)), ((), ())),
            preferred_element_type=jnp.float32)
        out_ref[...] = pooled / denom + bm_ref[...]


def _make_call(G, Nb, B, N, S, D):
    return pl.pallas_call(
        functools.partial(_body, G, Nb, B, N, S),
        grid=(G,),
        in_specs=[
            pl.BlockSpec(memory_space=pltpu.SMEM),
            pl.BlockSpec((Nb, D), lambda i: (i, 0)),
            pl.BlockSpec((1, 1, Nb), lambda i: (i, 0, 0)),
            pl.BlockSpec((1, 1, Nb), lambda i: (i, 0, 0)),
            pl.BlockSpec((D, 1), lambda i: (0, 0)),
            pl.BlockSpec((1, 1), lambda i: (0, 0)),
            pl.BlockSpec((D, D), lambda i: (0, 0)),
            pl.BlockSpec((1, D), lambda i: (0, 0)),
        ],
        out_specs=pl.BlockSpec((B, D), lambda i: (0, 0)),
        out_shape=jax.ShapeDtypeStruct((B, D), jnp.float32),
        scratch_shapes=[pltpu.VMEM((B, D), jnp.float32),
                        pltpu.VMEM((B, 1), jnp.float32)],
    )


def kernel(x, edge_index, pos, batch_index, W_gate, b_gate, W_msg, b_msg):
    del edge_index
    N, D = x.shape
    B = _NUM_SEGMENTS
    S = _WIN
    Nb = 5000 if N % 5000 == 0 else 1024
    G = -(-N // Nb)
    pad = G * Nb - N
    if pad:
        pos = jnp.pad(pos, (0, pad))
        batch_index = jnp.pad(batch_index, (0, pad), mode="edge")
    bid_r = batch_index.reshape(G, Nb)
    firsts = bid_r[:, 0]
    lasts = bid_r[:, -1]
    bases = jnp.minimum((firsts // 8) * 8, B - S)
    all_small = jnp.all(lasts - bases < S)
    pos3 = pos.reshape(G, 1, Nb)
    bid3 = batch_index.reshape(G, 1, Nb)
    bg2 = b_gate.reshape(1, 1)
    bm2 = b_msg.reshape(1, D)
    ops = (bases, x, pos3, bid3, W_gate, bg2, W_msg, bm2)
    return jax.lax.cond(
        all_small,
        lambda o: _make_call(G, Nb, B, N, S, D)(*o),
        lambda o: _make_call(G, Nb, B, N, B, D)(*o),
        ops)
